# baseline (speedup 1.0000x reference)
import functools

import jax
import jax.numpy as jnp
from jax import lax
from jax.experimental import pallas as pl
from jax.experimental.pallas import tpu as pltpu
from jax.experimental.pallas import tpu_sc as plsc

N = 160000
D = 256
NB = 96
DUMP = 93
CH = 128
NS = 16
NC = 2
COLS = D // NC
RPT = 8
NFT = NB // RPT
L = 16
NK = 64
TROWS = NK * CH
ROWS_SC = NS * TROWS
ROWS_TC = N - ROWS_SC
CT = 256
GT = ROWS_TC // CT
BPG = 8


def _sc_segment_sum(features, labels):
    mesh = plsc.VectorSubcoreMesh(core_axis_name="c", subcore_axis_name="s")

    @functools.partial(
        pl.kernel,
        out_type=jax.ShapeDtypeStruct((NB, D), jnp.float32),
        mesh=mesh,
        scratch_types=[
            pltpu.VMEM((CH, COLS), jnp.float32),
            pltpu.VMEM((CH, COLS), jnp.float32),
            pltpu.VMEM((CH, COLS), jnp.float32),
            pltpu.VMEM((CH, COLS), jnp.float32),
            pltpu.VMEM((TROWS,), jnp.int32),
            pltpu.VMEM((RPT, COLS), jnp.float32),
            pltpu.VMEM_SHARED((NB, COLS), jnp.float32),
            pltpu.SemaphoreType.DMA,
            pltpu.SemaphoreType.DMA,
            pltpu.SemaphoreType.DMA,
            pltpu.SemaphoreType.DMA,
            pltpu.SemaphoreType.DMA,
            pltpu.SemaphoreType.DMA,
            pltpu.SemaphoreType.DMA,
            pltpu.SemaphoreType.DMA,
            pltpu.SemaphoreType.DMA,
        ],
    )
    def body(feat_hbm, lab_hbm, out_hbm, f0, f1, f2, f3, lab_all,
             tmp, accum, sf0, sf1, sf2, sf3, ss0, ss1, ss2, ss3, sl):
        cid = lax.axis_index("c")
        sid = lax.axis_index("s")
        col0 = pl.multiple_of(cid * COLS, COLS)
        r0 = pl.multiple_of(sid * RPT, RPT)
        tbase = pl.multiple_of(sid * TROWS, CH)

        zero16 = jnp.zeros((L,), jnp.float32)
        for i in range(RPT):
            for k in range(COLS // L):
                tmp[i, pl.ds(k * L, L)] = zero16

        @pl.when(sid < NFT)
        def _init():
            pltpu.sync_copy(tmp, accum.at[pl.ds(r0, RPT), :])

        pltpu.async_copy(lab_hbm.at[pl.ds(tbase, TROWS)], lab_all, sl)

        plsc.subcore_barrier()
        pltpu.make_async_copy(lab_hbm.at[pl.ds(tbase, TROWS)], lab_all,
                              sl).wait()

        def remap(i, carry):
            v = lab_all[pl.ds(i * L, L)]
            ids = v - 3
            ids = jnp.where((ids < 0) | (ids > DUMP - 1), DUMP, ids)
            lab_all[pl.ds(i * L, L)] = ids
            return carry

        lax.fori_loop(0, TROWS // L, remap, 0)

        def feat_src(k):
            base = pl.multiple_of((sid * NK + k) * CH, CH)
            return feat_hbm.at[pl.ds(base, CH), pl.ds(col0, COLS)]

        def issue_in(k, fb, sf):
            pltpu.async_copy(feat_src(k), fb, sf)

        def wait_in(k, fb, sf):
            pltpu.make_async_copy(feat_src(k), fb, sf).wait()

        def ids_at(k):
            return lab_all.at[pl.ds(k * CH, CH)]

        def issue_sc(fb, k, ss):
            pltpu.async_copy(fb, accum.at[ids_at(k)], ss, add=True)

        def wait_sc(fb, k, ss):
            pltpu.make_async_copy(fb, accum.at[ids_at(k)], ss).wait()

        issue_in(0, f0, sf0)
        issue_in(1, f1, sf1)
        issue_in(2, f2, sf2)
        issue_in(3, f3, sf3)

        def quadbody(kk, carry):
            k = 4 * kk
            wait_in(k, f0, sf0)
            issue_sc(f0, k, ss0)
            wait_in(k + 1, f1, sf1)
            issue_sc(f1, k + 1, ss1)
            wait_in(k + 2, f2, sf2)
            issue_sc(f2, k + 2, ss2)
            wait_in(k + 3, f3, sf3)
            issue_sc(f3, k + 3, ss3)
            wait_sc(f0, k, ss0)
            issue_in(k + 4, f0, sf0)
            wait_sc(f1, k + 1, ss1)
            issue_in(k + 5, f1, sf1)
            wait_sc(f2, k + 2, ss2)
            issue_in(k + 6, f2, sf2)
            wait_sc(f3, k + 3, ss3)
            issue_in(k + 7, f3, sf3)
            return carry

        lax.fori_loop(0, (NK - 4) // 4, quadbody, 0)
        wait_in(NK - 4, f0, sf0)
        issue_sc(f0, NK - 4, ss0)
        wait_in(NK - 3, f1, sf1)
        issue_sc(f1, NK - 3, ss1)
        wait_in(NK - 2, f2, sf2)
        issue_sc(f2, NK - 2, ss2)
        wait_in(NK - 1, f3, sf3)
        issue_sc(f3, NK - 1, ss3)
        wait_sc(f0, NK - 4, ss0)
        wait_sc(f1, NK - 3, ss1)
        wait_sc(f2, NK - 2, ss2)
        wait_sc(f3, NK - 1, ss3)

        plsc.subcore_barrier()

        @pl.when(sid < NFT)
        def _finalize():
            pltpu.sync_copy(accum.at[pl.ds(r0, RPT), :],
                            out_hbm.at[pl.ds(r0, RPT), pl.ds(col0, COLS)])

    return body(features, labels)


def _histo_kernel(lab_ref, out_ref):
    g = pl.program_id(0)
    lab = lab_ref[...]
    rows = []
    for i in range(BPG):
        b = g * BPG + i
        cnt = jnp.sum((lab == (b + 3)).astype(jnp.float32))
        rows.append(jnp.full((D,), cnt, jnp.float32))
    out_ref[...] = jnp.stack(rows, axis=0)


def _tc_histogram(labels2d):
    return pl.pallas_call(
        _histo_kernel,
        grid=(NB // BPG,),
        in_specs=[pl.BlockSpec(labels2d.shape, lambda g: (0, 0))],
        out_specs=pl.BlockSpec((BPG, D), lambda g: (g, 0)),
        out_shape=jax.ShapeDtypeStruct((NB, D), jnp.float32),
    )(labels2d)


def _psum_kernel(lab_ref, feat_ref, out_ref):
    lab = lab_ref[...]
    feat = feat_ref[...]
    buckets = lax.broadcasted_iota(jnp.int32, (NB, CT), 0) + 3
    onehot = (buckets == lab).astype(jnp.float32)
    part = jax.lax.dot_general(
        onehot, feat, (((1,), (0,)), ((), ())),
        precision=lax.Precision.HIGHEST,
        preferred_element_type=jnp.float32)

    @pl.when(pl.program_id(0) == 0)
    def _():
        out_ref[...] = jnp.zeros_like(out_ref)

    out_ref[...] += part


def _tc_partial_sum(features, labels2d):
    return pl.pallas_call(
        _psum_kernel,
        grid=(GT,),
        in_specs=[
            pl.BlockSpec((1, CT), lambda g: (0, ROWS_SC // CT + g)),
            pl.BlockSpec((CT, D), lambda g: (ROWS_SC // CT + g, 0)),
        ],
        out_specs=pl.BlockSpec((NB, D), lambda g: (0, 0)),
        out_shape=jax.ShapeDtypeStruct((NB, D), jnp.float32),
    )(labels2d, features)


def _div_kernel(sum_ref, sum2_ref, cnt_ref, out_ref):
    s = sum_ref[...] + sum2_ref[...]
    c = cnt_ref[...]
    out_ref[...] = jnp.where(c > 0.0, s / jnp.where(c > 0.0, c, 1.0), 0.0)


def _tc_divide(sums, sums2, cnts):
    return pl.pallas_call(
        _div_kernel,
        in_specs=[pl.BlockSpec(sums.shape, lambda: (0, 0)),
                  pl.BlockSpec(sums2.shape, lambda: (0, 0)),
                  pl.BlockSpec(cnts.shape, lambda: (0, 0))],
        out_specs=pl.BlockSpec(sums.shape, lambda: (0, 0)),
        out_shape=jax.ShapeDtypeStruct(sums.shape, jnp.float32),
    )(sums, sums2, cnts)


def kernel(features, labels):
    sums_sc = _sc_segment_sum(features, labels)
    sums_tc = _tc_partial_sum(features, labels.reshape(1, N))
    cnts = _tc_histogram(labels.reshape(N // CH, CH))
    center = _tc_divide(sums_sc, sums_tc, cnts)
    return center[:DUMP]


# speedup vs baseline: 1.0835x; 1.0835x over previous
import functools

import jax
import jax.numpy as jnp
from jax import lax
from jax.experimental import pallas as pl
from jax.experimental.pallas import tpu as pltpu
from jax.experimental.pallas import tpu_sc as plsc

N = 160000
D = 256
NB = 96
DUMP = 93
CH = 128
NS = 16
NC = 2
COLS = D // NC
RPT = 8
NFT = NB // RPT
L = 16
NK = 68
TROWS = NK * CH
ROWS_SC = NS * TROWS
ROWS_TC = N - ROWS_SC
CT = 256
GT = ROWS_TC // CT
BPG = 8


def _sc_segment_sum(features, labels):
    mesh = plsc.VectorSubcoreMesh(core_axis_name="c", subcore_axis_name="s")

    @functools.partial(
        pl.kernel,
        out_type=jax.ShapeDtypeStruct((NB, D), jnp.float32),
        mesh=mesh,
        scratch_types=[
            pltpu.VMEM((CH, COLS), jnp.float32),
            pltpu.VMEM((CH, COLS), jnp.float32),
            pltpu.VMEM((CH, COLS), jnp.float32),
            pltpu.VMEM((CH, COLS), jnp.float32),
            pltpu.VMEM((TROWS,), jnp.int32),
            pltpu.VMEM((RPT, COLS), jnp.float32),
            pltpu.VMEM_SHARED((NB, COLS), jnp.float32),
            pltpu.SemaphoreType.DMA,
            pltpu.SemaphoreType.DMA,
            pltpu.SemaphoreType.DMA,
            pltpu.SemaphoreType.DMA,
            pltpu.SemaphoreType.DMA,
            pltpu.SemaphoreType.DMA,
            pltpu.SemaphoreType.DMA,
            pltpu.SemaphoreType.DMA,
            pltpu.SemaphoreType.DMA,
        ],
    )
    def body(feat_hbm, lab_hbm, out_hbm, f0, f1, f2, f3, lab_all,
             tmp, accum, sf0, sf1, sf2, sf3, ss0, ss1, ss2, ss3, sl):
        cid = lax.axis_index("c")
        sid = lax.axis_index("s")
        col0 = pl.multiple_of(cid * COLS, COLS)
        r0 = pl.multiple_of(sid * RPT, RPT)
        tbase = pl.multiple_of(sid * TROWS, CH)

        zero16 = jnp.zeros((L,), jnp.float32)
        for i in range(RPT):
            for k in range(COLS // L):
                tmp[i, pl.ds(k * L, L)] = zero16

        @pl.when(sid < NFT)
        def _init():
            pltpu.sync_copy(tmp, accum.at[pl.ds(r0, RPT), :])

        pltpu.async_copy(lab_hbm.at[pl.ds(tbase, TROWS)], lab_all, sl)

        plsc.subcore_barrier()
        pltpu.make_async_copy(lab_hbm.at[pl.ds(tbase, TROWS)], lab_all,
                              sl).wait()

        def remap(i, carry):
            v = lab_all[pl.ds(i * L, L)]
            ids = v - 3
            ids = jnp.where((ids < 0) | (ids > DUMP - 1), DUMP, ids)
            lab_all[pl.ds(i * L, L)] = ids
            return carry

        lax.fori_loop(0, TROWS // L, remap, 0)

        def feat_src(k):
            base = pl.multiple_of((sid * NK + k) * CH, CH)
            return feat_hbm.at[pl.ds(base, CH), pl.ds(col0, COLS)]

        def issue_in(k, fb, sf):
            pltpu.async_copy(feat_src(k), fb, sf)

        def wait_in(k, fb, sf):
            pltpu.make_async_copy(feat_src(k), fb, sf).wait()

        def ids_at(k):
            return lab_all.at[pl.ds(k * CH, CH)]

        def issue_sc(fb, k, ss):
            pltpu.async_copy(fb, accum.at[ids_at(k)], ss, add=True)

        def wait_sc(fb, k, ss):
            pltpu.make_async_copy(fb, accum.at[ids_at(k)], ss).wait()

        issue_in(0, f0, sf0)
        issue_in(1, f1, sf1)
        issue_in(2, f2, sf2)
        issue_in(3, f3, sf3)

        def quadbody(kk, carry):
            k = 4 * kk
            wait_in(k, f0, sf0)
            issue_sc(f0, k, ss0)
            wait_in(k + 1, f1, sf1)
            issue_sc(f1, k + 1, ss1)
            wait_in(k + 2, f2, sf2)
            issue_sc(f2, k + 2, ss2)
            wait_in(k + 3, f3, sf3)
            issue_sc(f3, k + 3, ss3)
            wait_sc(f0, k, ss0)
            issue_in(k + 4, f0, sf0)
            wait_sc(f1, k + 1, ss1)
            issue_in(k + 5, f1, sf1)
            wait_sc(f2, k + 2, ss2)
            issue_in(k + 6, f2, sf2)
            wait_sc(f3, k + 3, ss3)
            issue_in(k + 7, f3, sf3)
            return carry

        lax.fori_loop(0, (NK - 4) // 4, quadbody, 0)
        wait_in(NK - 4, f0, sf0)
        issue_sc(f0, NK - 4, ss0)
        wait_in(NK - 3, f1, sf1)
        issue_sc(f1, NK - 3, ss1)
        wait_in(NK - 2, f2, sf2)
        issue_sc(f2, NK - 2, ss2)
        wait_in(NK - 1, f3, sf3)
        issue_sc(f3, NK - 1, ss3)
        wait_sc(f0, NK - 4, ss0)
        wait_sc(f1, NK - 3, ss1)
        wait_sc(f2, NK - 2, ss2)
        wait_sc(f3, NK - 1, ss3)

        plsc.subcore_barrier()

        @pl.when(sid < NFT)
        def _finalize():
            pltpu.sync_copy(accum.at[pl.ds(r0, RPT), :],
                            out_hbm.at[pl.ds(r0, RPT), pl.ds(col0, COLS)])

    return body(features, labels)


def _histo_kernel(lab_ref, out_ref):
    g = pl.program_id(0)
    lab = lab_ref[...]
    rows = []
    for i in range(BPG):
        b = g * BPG + i
        cnt = jnp.sum((lab == (b + 3)).astype(jnp.float32))
        rows.append(jnp.full((D,), cnt, jnp.float32))
    out_ref[...] = jnp.stack(rows, axis=0)


def _tc_histogram(labels2d):
    return pl.pallas_call(
        _histo_kernel,
        grid=(NB // BPG,),
        in_specs=[pl.BlockSpec(labels2d.shape, lambda g: (0, 0))],
        out_specs=pl.BlockSpec((BPG, D), lambda g: (g, 0)),
        out_shape=jax.ShapeDtypeStruct((NB, D), jnp.float32),
    )(labels2d)


def _psum_kernel(lab_ref, feat_ref, out_ref):
    lab = lab_ref[...]
    feat = feat_ref[...]
    buckets = lax.broadcasted_iota(jnp.int32, (NB, CT), 0) + 3
    onehot = (buckets == lab).astype(jnp.float32)
    part = jax.lax.dot_general(
        onehot, feat, (((1,), (0,)), ((), ())),
        precision=lax.Precision.HIGHEST,
        preferred_element_type=jnp.float32)

    @pl.when(pl.program_id(0) == 0)
    def _():
        out_ref[...] = jnp.zeros_like(out_ref)

    out_ref[...] += part


def _tc_partial_sum(features, labels2d):
    return pl.pallas_call(
        _psum_kernel,
        grid=(GT,),
        in_specs=[
            pl.BlockSpec((1, CT), lambda g: (0, ROWS_SC // CT + g)),
            pl.BlockSpec((CT, D), lambda g: (ROWS_SC // CT + g, 0)),
        ],
        out_specs=pl.BlockSpec((NB, D), lambda g: (0, 0)),
        out_shape=jax.ShapeDtypeStruct((NB, D), jnp.float32),
    )(labels2d, features)


def _div_kernel(sum_ref, sum2_ref, cnt_ref, out_ref):
    s = sum_ref[...] + sum2_ref[...]
    c = cnt_ref[...]
    out_ref[...] = jnp.where(c > 0.0, s / jnp.where(c > 0.0, c, 1.0), 0.0)


def _tc_divide(sums, sums2, cnts):
    return pl.pallas_call(
        _div_kernel,
        in_specs=[pl.BlockSpec(sums.shape, lambda: (0, 0)),
                  pl.BlockSpec(sums2.shape, lambda: (0, 0)),
                  pl.BlockSpec(cnts.shape, lambda: (0, 0))],
        out_specs=pl.BlockSpec(sums.shape, lambda: (0, 0)),
        out_shape=jax.ShapeDtypeStruct(sums.shape, jnp.float32),
    )(sums, sums2, cnts)


def kernel(features, labels):
    sums_sc = _sc_segment_sum(features, labels)
    sums_tc = _tc_partial_sum(features, labels.reshape(1, N))
    cnts = _tc_histogram(labels.reshape(N // CH, CH))
    center = _tc_divide(sums_sc, sums_tc, cnts)
    return center[:DUMP]
